# baseline (speedup 1.0000x reference)
import jax
import jax.numpy as jnp
from jax.experimental import pallas as pl
from jax.experimental.pallas import tpu as pltpu

_CIN, _H, _W = 64, 56, 56
_HW = _H * _W
_WO = 28
_HWL = _H * _WO
_CH = 32
_CL = 32
_PAD = 64
_SROWS = 3264
_TROWS = 1792


def _octave_kernel(x_ref, w_h_ref, w_l_ref, i64_ref,
                   o_h_ref, o_l_ref, sc_ref, sl_ref, sr_ref,
                   r_ref, tc_ref, tl_ref, tr_ref):
    f32 = jnp.float32

    xt = jax.lax.dot_general(x_ref[...], i64_ref[...],
                             (((0,), (0,)), ((), ())),
                             preferred_element_type=f32)

    def fill(c_ref, l_ref, rr_ref, val, n_rows, w):
        kills = n_rows // w
        for ref, base in ((c_ref, _PAD), (l_ref, _PAD + 1), (rr_ref, _PAD - 1)):
            ref[pl.ds(0, _PAD + 1), :] = jnp.zeros((_PAD + 1, _CIN), f32)
            ref[pl.ds(base + n_rows, 63), :] = jnp.zeros((63, _CIN), f32)
            ref[pl.ds(base, n_rows), :] = val
        l_ref[pl.ds(_PAD + 1 + w - 1, kills, stride=w), :] = (
            jnp.zeros((kills, _CIN), f32))
        rr_ref[pl.ds(_PAD - 1, kills, stride=w), :] = (
            jnp.zeros((kills, _CIN), f32))

    fill(sc_ref, sl_ref, sr_ref, xt, _HW, _W)

    x = x_ref[...]
    r = (x + pltpu.roll(x, _HW - 1, 1) + pltpu.roll(x, _HW - _W, 1)
         + pltpu.roll(x, _HW - _W - 1, 1))
    r_ref[...] = jax.lax.dot_general(r, i64_ref[...], (((0,), (0,)), ((), ())),
                                     preferred_element_type=f32)
    u = r_ref[pl.ds(0, _HWL, stride=2), :]
    fill(tc_ref, tl_ref, tr_ref, u, _HWL, _WO)

    def conv(refs, n_rows, step, w_ref):
        taps = []
        for kh in range(3):
            for kw in range(3):
                o = _PAD + step * (kh - 1)
                taps.append(refs[kw][pl.ds(o, n_rows), :])
        cols = jnp.concatenate(taps, axis=-1)
        return jnp.dot(cols, w_ref[...], preferred_element_type=f32)

    out_h = conv((sl_ref, sc_ref, sr_ref), _HW, _W, w_h_ref)
    out_l = conv((tl_ref, tc_ref, tr_ref), _HWL, 2 * _WO, w_l_ref)

    i32 = i64_ref[pl.ds(0, _CH), pl.ds(0, _CH)]
    o_h_ref[...] = jax.lax.dot_general(i32, out_h, (((1,), (1,)), ((), ())),
                                       preferred_element_type=f32)
    o_l_ref[...] = jax.lax.dot_general(i32, out_l, (((1,), (1,)), ((), ())),
                                       preferred_element_type=f32)


def _pack_weight(w_oihw, scale=None):
    o, i, kh, kw = w_oihw.shape
    w = jnp.transpose(w_oihw, (2, 3, 1, 0)).reshape(kh * kw * i, o)
    if scale is not None:
        w = w * scale
    return w


def kernel(x_nchw, w_h2h_oihw, w_h2l_oihw):
    n = x_nchw.shape[0]
    x_flat = x_nchw.reshape(n, _CIN, _HW)
    w_h = _pack_weight(w_h2h_oihw)
    w_l = _pack_weight(w_h2l_oihw, scale=0.25)
    i64 = jnp.eye(_CIN, dtype=jnp.float32)

    out_h, out_l = pl.pallas_call(
        _octave_kernel,
        out_shape=(jax.ShapeDtypeStruct((n, _CH, _HW), jnp.float32),
                   jax.ShapeDtypeStruct((n, _CL, _HWL), jnp.float32)),
        grid_spec=pltpu.PrefetchScalarGridSpec(
            num_scalar_prefetch=0,
            grid=(n,),
            in_specs=[
                pl.BlockSpec((None, _CIN, _HW), lambda i: (i, 0, 0)),
                pl.BlockSpec((9 * _CIN, _CH), lambda i: (0, 0)),
                pl.BlockSpec((9 * _CIN, _CL), lambda i: (0, 0)),
                pl.BlockSpec((_CIN, _CIN), lambda i: (0, 0)),
            ],
            out_specs=[
                pl.BlockSpec((None, _CH, _HW), lambda i: (i, 0, 0)),
                pl.BlockSpec((None, _CL, _HWL), lambda i: (i, 0, 0)),
            ],
            scratch_shapes=[
                pltpu.VMEM((_SROWS, _CIN), jnp.float32),
                pltpu.VMEM((_SROWS, _CIN), jnp.float32),
                pltpu.VMEM((_SROWS, _CIN), jnp.float32),
                pltpu.VMEM((_HW, _CIN), jnp.float32),
                pltpu.VMEM((_TROWS, _CIN), jnp.float32),
                pltpu.VMEM((_TROWS, _CIN), jnp.float32),
                pltpu.VMEM((_TROWS, _CIN), jnp.float32),
            ],
        ),
        compiler_params=pltpu.CompilerParams(
            dimension_semantics=("parallel",),
            vmem_limit_bytes=64 * 1024 * 1024,
        ),
    )(x_flat, w_h, w_l, i64)

    x_h = out_h.reshape(n, _CH, _H, _W)
    x_l = out_l.reshape(n, _CL, _H, _WO)[:, :, 0::2, :]
    return x_h, x_l


# speedup vs baseline: 1.3899x; 1.3899x over previous
import jax
import jax.numpy as jnp
from jax.experimental import pallas as pl
from jax.experimental.pallas import tpu as pltpu

_CIN, _H, _W = 64, 56, 56
_HW = _H * _W
_HO, _WO = 28, 28
_HWL = _H * _WO
_CH = 32
_CL = 32
_PAD = 64
_SROWS = 3264
_TROWS = 1792


def _octave_kernel(x_ref, w_h_ref, w_l_ref, i64_ref,
                   o_h_ref, o_l_ref, sc_ref, sl_ref, sr_ref,
                   r_ref, tc_ref, tl_ref, tr_ref, y_ref):
    f32 = jnp.float32

    xt = jax.lax.dot_general(x_ref[...], i64_ref[...],
                             (((0,), (0,)), ((), ())),
                             preferred_element_type=f32)

    def fill(c_ref, l_ref, rr_ref, val, n_rows, w):
        kills = n_rows // w
        for ref, base in ((c_ref, _PAD), (l_ref, _PAD + 1), (rr_ref, _PAD - 1)):
            ref[pl.ds(0, _PAD + 1), :] = jnp.zeros((_PAD + 1, _CIN), f32)
            ref[pl.ds(base + n_rows, 63), :] = jnp.zeros((63, _CIN), f32)
            ref[pl.ds(base, n_rows), :] = val
        l_ref[pl.ds(_PAD + 1 + w - 1, kills, stride=w), :] = (
            jnp.zeros((kills, _CIN), f32))
        rr_ref[pl.ds(_PAD - 1, kills, stride=w), :] = (
            jnp.zeros((kills, _CIN), f32))

    fill(sc_ref, sl_ref, sr_ref, xt, _HW, _W)

    x = x_ref[...]
    r = (x + pltpu.roll(x, _HW - 1, 1) + pltpu.roll(x, _HW - _W, 1)
         + pltpu.roll(x, _HW - _W - 1, 1))
    r_ref[...] = jax.lax.dot_general(r, i64_ref[...], (((0,), (0,)), ((), ())),
                                     preferred_element_type=f32)
    u = r_ref[pl.ds(0, _HWL, stride=2), :]
    fill(tc_ref, tl_ref, tr_ref, u, _HWL, _WO)

    def conv(refs, n_rows, step, w_ref):
        acc = jnp.zeros((n_rows, _CH), f32)
        for kh in range(3):
            o = _PAD + step * (kh - 1)
            for kw in range(3):
                tap = refs[kw][pl.ds(o, n_rows), :]
                wk = w_ref[pl.ds((kh * 3 + kw) * _CIN, _CIN), :]
                acc = acc + jnp.dot(tap, wk, preferred_element_type=f32)
        return acc

    out_h = conv((sl_ref, sc_ref, sr_ref), _HW, _W, w_h_ref)
    out_l = conv((tl_ref, tc_ref, tr_ref), _HWL, 2 * _WO, w_l_ref)

    i32 = i64_ref[pl.ds(0, _CH), pl.ds(0, _CH)]
    hcm = jax.lax.dot_general(i32, out_h, (((1,), (1,)), ((), ())),
                              preferred_element_type=f32)
    o_h_ref[...] = hcm.reshape(_CH, _H, _W)
    lcm = jax.lax.dot_general(i32, out_l, (((1,), (1,)), ((), ())),
                              preferred_element_type=f32)
    y_ref[...] = lcm.reshape(_CL, _H, _WO)
    o_l_ref[...] = y_ref[:, pl.ds(0, _HO, stride=2), :]


def _pack_weight(w_oihw, scale=None):
    o, i, kh, kw = w_oihw.shape
    w = jnp.transpose(w_oihw, (2, 3, 1, 0)).reshape(kh * kw * i, o)
    if scale is not None:
        w = w * scale
    return w


def kernel(x_nchw, w_h2h_oihw, w_h2l_oihw):
    n = x_nchw.shape[0]
    x_flat = x_nchw.reshape(n, _CIN, _HW)
    w_h = _pack_weight(w_h2h_oihw)
    w_l = _pack_weight(w_h2l_oihw, scale=0.25)
    i64 = jnp.eye(_CIN, dtype=jnp.float32)

    x_h, x_l = pl.pallas_call(
        _octave_kernel,
        out_shape=(jax.ShapeDtypeStruct((n, _CH, _H, _W), jnp.float32),
                   jax.ShapeDtypeStruct((n, _CL, _HO, _WO), jnp.float32)),
        grid_spec=pltpu.PrefetchScalarGridSpec(
            num_scalar_prefetch=0,
            grid=(n,),
            in_specs=[
                pl.BlockSpec((None, _CIN, _HW), lambda i: (i, 0, 0)),
                pl.BlockSpec((9 * _CIN, _CH), lambda i: (0, 0)),
                pl.BlockSpec((9 * _CIN, _CL), lambda i: (0, 0)),
                pl.BlockSpec((_CIN, _CIN), lambda i: (0, 0)),
            ],
            out_specs=[
                pl.BlockSpec((None, _CH, _H, _W), lambda i: (i, 0, 0, 0)),
                pl.BlockSpec((None, _CL, _HO, _WO), lambda i: (i, 0, 0, 0)),
            ],
            scratch_shapes=[
                pltpu.VMEM((_SROWS, _CIN), jnp.float32),
                pltpu.VMEM((_SROWS, _CIN), jnp.float32),
                pltpu.VMEM((_SROWS, _CIN), jnp.float32),
                pltpu.VMEM((_HW, _CIN), jnp.float32),
                pltpu.VMEM((_TROWS, _CIN), jnp.float32),
                pltpu.VMEM((_TROWS, _CIN), jnp.float32),
                pltpu.VMEM((_TROWS, _CIN), jnp.float32),
                pltpu.VMEM((_CL, _H, _WO), jnp.float32),
            ],
        ),
        compiler_params=pltpu.CompilerParams(
            dimension_semantics=("parallel",),
            vmem_limit_bytes=64 * 1024 * 1024,
        ),
    )(x_flat, w_h, w_l, i64)

    return x_h, x_l


# speedup vs baseline: 2.3569x; 1.6958x over previous
import jax
import jax.numpy as jnp
from jax.experimental import pallas as pl
from jax.experimental.pallas import tpu as pltpu

_CIN, _H, _W = 64, 56, 56
_HW = _H * _W
_HO, _WO = 28, 28
_HWL = _HO * _WO
_CH = 32
_CL = 32
_PAD = 64
_SROWS = 3264
_TROWS = 912


def _octave_kernel(x_ref, w_h_ref, w_l_ref, o_h_ref, o_l_ref,
                   sc_ref, sl_ref, sr_ref, tc_ref, tl_ref, tr_ref):
    f32 = jnp.float32
    bf16 = jnp.bfloat16

    def fill(c_ref, l_ref, rr_ref, val, n_rows, w):
        kills = n_rows // w
        for ref, base in ((c_ref, _PAD), (l_ref, _PAD + 1), (rr_ref, _PAD - 1)):
            ref[pl.ds(0, _PAD + 1), :] = jnp.zeros((_PAD + 1, _CIN), f32)
            ref[pl.ds(base + n_rows, 63), :] = jnp.zeros((63, _CIN), f32)
            ref[pl.ds(base, n_rows), :] = val
        l_ref[pl.ds(_PAD + 1 + w - 1, kills, stride=w), :] = (
            jnp.zeros((kills, _CIN), f32))
        rr_ref[pl.ds(_PAD - 1, kills, stride=w), :] = (
            jnp.zeros((kills, _CIN), f32))

    fill(sc_ref, sl_ref, sr_ref, x_ref[...].reshape(_HW, _CIN), _HW, _W)

    pooled = (x_ref[pl.ds(0, _HO, stride=2), pl.ds(0, _WO, stride=2), :]
              + x_ref[pl.ds(0, _HO, stride=2), pl.ds(1, _WO, stride=2), :]
              + x_ref[pl.ds(1, _HO, stride=2), pl.ds(0, _WO, stride=2), :]
              + x_ref[pl.ds(1, _HO, stride=2), pl.ds(1, _WO, stride=2), :])
    fill(tc_ref, tl_ref, tr_ref, pooled.reshape(_HWL, _CIN), _HWL, _WO)

    def conv(refs, n_rows, step, w_ref):
        taps = []
        for kh in range(3):
            o = _PAD + step * (kh - 1)
            for kw in range(3):
                taps.append(refs[kw][pl.ds(o, n_rows), :])
        cols = jnp.concatenate(taps, axis=-1).astype(bf16)
        return jnp.dot(cols, w_ref[...], preferred_element_type=f32)

    o_h_ref[...] = conv((sl_ref, sc_ref, sr_ref), _HW, _W, w_h_ref)
    o_l_ref[...] = conv((tl_ref, tc_ref, tr_ref), _HWL, _WO, w_l_ref)


def _pack_weight(w_oihw, scale=None):
    o, i, kh, kw = w_oihw.shape
    w = jnp.transpose(w_oihw, (2, 3, 1, 0)).reshape(kh * kw * i, o)
    if scale is not None:
        w = w * scale
    return w.astype(jnp.bfloat16)


def kernel(x_nchw, w_h2h_oihw, w_h2l_oihw):
    n = x_nchw.shape[0]
    x_nhwc = jnp.transpose(x_nchw, (0, 2, 3, 1))
    w_h = _pack_weight(w_h2h_oihw)
    w_l = _pack_weight(w_h2l_oihw, scale=0.25)

    out_h, out_l = pl.pallas_call(
        _octave_kernel,
        out_shape=(jax.ShapeDtypeStruct((n, _HW, _CH), jnp.float32),
                   jax.ShapeDtypeStruct((n, _HWL, _CL), jnp.float32)),
        grid_spec=pltpu.PrefetchScalarGridSpec(
            num_scalar_prefetch=0,
            grid=(n,),
            in_specs=[
                pl.BlockSpec((None, _H, _W, _CIN), lambda i: (i, 0, 0, 0)),
                pl.BlockSpec((9 * _CIN, _CH), lambda i: (0, 0)),
                pl.BlockSpec((9 * _CIN, _CL), lambda i: (0, 0)),
            ],
            out_specs=[
                pl.BlockSpec((None, _HW, _CH), lambda i: (i, 0, 0)),
                pl.BlockSpec((None, _HWL, _CL), lambda i: (i, 0, 0)),
            ],
            scratch_shapes=[
                pltpu.VMEM((_SROWS, _CIN), jnp.float32),
                pltpu.VMEM((_SROWS, _CIN), jnp.float32),
                pltpu.VMEM((_SROWS, _CIN), jnp.float32),
                pltpu.VMEM((_TROWS, _CIN), jnp.float32),
                pltpu.VMEM((_TROWS, _CIN), jnp.float32),
                pltpu.VMEM((_TROWS, _CIN), jnp.float32),
            ],
        ),
        compiler_params=pltpu.CompilerParams(
            dimension_semantics=("parallel",),
            vmem_limit_bytes=64 * 1024 * 1024,
        ),
    )(x_nhwc, w_h, w_l)

    x_h = jnp.transpose(out_h.reshape(n, _H, _W, _CH), (0, 3, 1, 2))
    x_l = jnp.transpose(out_l.reshape(n, _HO, _WO, _CL), (0, 3, 1, 2))
    return x_h, x_l
